# Initial kernel scaffold; baseline (speedup 1.0000x reference)
import jax
import jax.numpy as jnp
from jax.experimental import pallas as pl


def kernel(x, conv_w, conv_b, pos_emb, ln1_g, ln1_b, wq, bq, wk, bk, wv, bv, wo, bo, ln2_g, ln2_b, gr_w, gr_b, er_w, er_b, w1, b1, w2, b2, lnf_g, lnf_b, head_w, head_b):
    raise NotImplementedError("write your pallas kernel here")


# speedup vs baseline: 2.0609x; 2.0609x over previous
import jax
import jax.numpy as jnp
from jax import lax
from jax.experimental import pallas as pl
from jax.experimental.pallas import tpu as pltpu

B = 4; IN_CH = 80; T = 512; D = 256; DFF = 1024; G = 4; EPG = 4; E = 16
H = 4; DH = D // H; L = 2; S = 512; TOPK = 2; NCLS = 35
N = B * S
TILE = 128
P = N * TOPK + E * TILE
NT = P // TILE
KIN = 3 * IN_CH


def _gelu(v):
    return 0.5 * v * (1.0 + lax.erf(v * 0.7071067811865476))


def _layernorm(x, g, b):
    m = jnp.mean(x, axis=-1, keepdims=True)
    v = jnp.mean((x - m) ** 2, axis=-1, keepdims=True)
    return (x - m) * lax.rsqrt(v + 1e-5) * g + b


def _stem_body(x_ref, w_ref, b_ref, pos_ref, o_ref):
    y = jnp.dot(x_ref[...], w_ref[...], preferred_element_type=jnp.float32)
    o_ref[...] = _gelu(y + b_ref[...]) + pos_ref[...]


def _attn_body(h_ref, g_ref, bb_ref, wq_ref, bq_ref, wk_ref, bk_ref,
               wv_ref, bv_ref, wo_ref, bo_ref, o_ref):
    hh = h_ref[...]
    hn = _layernorm(hh, g_ref[...], bb_ref[...])
    q = jnp.dot(hn, wq_ref[...], preferred_element_type=jnp.float32) + bq_ref[...]
    k = jnp.dot(hn, wk_ref[...], preferred_element_type=jnp.float32) + bk_ref[...]
    v = jnp.dot(hn, wv_ref[...], preferred_element_type=jnp.float32) + bv_ref[...]
    s = lax.dot_general(q, k, (((1,), (1,)), ((), ())),
                        preferred_element_type=jnp.float32) * (DH ** -0.5)
    s = s - jnp.max(s, axis=-1, keepdims=True)
    e = jnp.exp(s)
    att = e / jnp.sum(e, axis=-1, keepdims=True)
    oh = jnp.dot(att, v, preferred_element_type=jnp.float32)
    part = jnp.dot(oh, wo_ref[...], preferred_element_type=jnp.float32)

    @pl.when(pl.program_id(1) == 0)
    def _init():
        o_ref[...] = hh + part + bo_ref[...]

    @pl.when(pl.program_id(1) != 0)
    def _acc():
        o_ref[...] += part


def _router_body(h_ref, g_ref, b_ref, wg_ref, bg_ref, we_ref, be_ref,
                 hn_ref, meta_ref):
    hn = _layernorm(h_ref[...], g_ref[...], b_ref[...])
    hn_ref[...] = hn
    gl = jnp.dot(hn, wg_ref[...], preferred_element_type=jnp.float32) + bg_ref[...]
    el = jnp.dot(hn, we_ref[...], preferred_element_type=jnp.float32) + be_ref[...]
    lane = lax.broadcasted_iota(jnp.int32, gl.shape, 1)
    neg = jnp.float32(-1e30)
    glm = jnp.where(lane < G, gl, neg)
    ge = jnp.where(lane < G, jnp.exp(glm - jnp.max(glm, -1, keepdims=True)), 0.0)
    gp = ge / jnp.sum(ge, -1, keepdims=True)
    elm = jnp.where(lane < E, el, neg)
    ee = jnp.where(lane < E, jnp.exp(elm - jnp.max(elm, -1, keepdims=True)), 0.0)
    r = lax.broadcasted_iota(jnp.int32, (128, 128), 0)
    c = lax.broadcasted_iota(jnp.int32, (128, 128), 1)
    grpsum = jnp.where((r < E) & (c < E) & ((r // EPG) == (c // EPG)), 1.0, 0.0)
    es = jnp.dot(ee, grpsum, preferred_element_type=jnp.float32,
                 precision=lax.Precision.HIGHEST)
    ep = ee / jnp.where(lane < E, es, 1.0)
    bcast = jnp.where((r < G) & (c < E) & ((c // EPG) == r), 1.0, 0.0)
    gpb = jnp.dot(gp, bcast, preferred_element_type=jnp.float32,
                  precision=lax.Precision.HIGHEST)
    comb = jnp.where(lane < E, gpb * ep, neg)
    big = jnp.int32(1 << 30)
    v1 = jnp.max(comb, -1, keepdims=True)
    a1 = jnp.min(jnp.where(comb == v1, lane, big), -1, keepdims=True)
    comb2 = jnp.where(lane == a1, neg, comb)
    v2 = jnp.max(comb2, -1, keepdims=True)
    a2 = jnp.min(jnp.where(comb2 == v2, lane, big), -1, keepdims=True)
    ssum = v1 + v2 + 1e-9
    meta = jnp.where(lane == 0, a1.astype(jnp.float32), 0.0)
    meta = meta + jnp.where(lane == 1, a2.astype(jnp.float32), 0.0)
    meta = meta + jnp.where(lane == 2, v1 / ssum, 0.0)
    meta = meta + jnp.where(lane == 3, v2 / ssum, 0.0)
    meta_ref[...] = meta


def _ffn_body(te_ref, x_ref, w1_ref, b1_ref, w2_ref, b2_ref, o_ref):
    h1 = _gelu(jnp.dot(x_ref[...], w1_ref[...],
                       preferred_element_type=jnp.float32) + b1_ref[...])
    o_ref[...] = jnp.dot(h1, w2_ref[...],
                         preferred_element_type=jnp.float32) + b2_ref[...]


def _head_body(h_ref, g_ref, b_ref, w_ref, bh_ref, o_ref):
    hh = h_ref[...]
    r = lax.broadcasted_iota(jnp.int32, (8, N), 0)
    c = lax.broadcasted_iota(jnp.int32, (8, N), 1)
    pool = jnp.where((c // S) == r, 1.0 / S, 0.0)
    p = jnp.dot(pool, hh, preferred_element_type=jnp.float32,
                precision=lax.Precision.HIGHEST)
    pn = _layernorm(p, g_ref[...], b_ref[...])
    o_ref[...] = jnp.dot(pn, w_ref[...],
                         preferred_element_type=jnp.float32) + bh_ref[...]


def kernel(x, conv_w, conv_b, pos_emb, ln1_g, ln1_b, wq, bq, wk, bk, wv, bv,
           wo, bo, ln2_g, ln2_b, gr_w, gr_b, er_w, er_b, w1, b1, w2, b2,
           lnf_g, lnf_b, head_w, head_b):
    f32 = jnp.float32

    xt = jnp.transpose(x, (0, 2, 1))
    xp = jnp.pad(xt, ((0, 0), (1, 1), (0, 0)))
    win = jnp.concatenate([xp[:, 0:T], xp[:, 1:T + 1], xp[:, 2:T + 2]], -1)
    xi = jnp.pad(win.reshape(N, KIN), ((0, 0), (0, 256 - KIN)))
    wmat = jnp.pad(conv_w.transpose(2, 1, 0).reshape(KIN, D),
                   ((0, 256 - KIN), (0, 0)))
    pos = jnp.tile(pos_emb[:S], (B, 1))
    h = pl.pallas_call(
        _stem_body,
        out_shape=jax.ShapeDtypeStruct((N, D), f32),
    )(xi, wmat, conv_b.reshape(1, D), pos)

    for l in range(L):
        h4 = h.reshape(B, S, D)
        wqh = wq[l].reshape(D, H, DH).transpose(1, 0, 2)
        wkh = wk[l].reshape(D, H, DH).transpose(1, 0, 2)
        wvh = wv[l].reshape(D, H, DH).transpose(1, 0, 2)
        woh = wo[l].reshape(H, DH, D)
        bqh = bq[l].reshape(H, 1, DH)
        bkh = bk[l].reshape(H, 1, DH)
        bvh = bv[l].reshape(H, 1, DH)
        h = pl.pallas_call(
            _attn_body,
            grid=(B, H),
            in_specs=[
                pl.BlockSpec((None, S, D), lambda b, hh: (b, 0, 0)),
                pl.BlockSpec((1, D), lambda b, hh: (0, 0)),
                pl.BlockSpec((1, D), lambda b, hh: (0, 0)),
                pl.BlockSpec((None, D, DH), lambda b, hh: (hh, 0, 0)),
                pl.BlockSpec((None, 1, DH), lambda b, hh: (hh, 0, 0)),
                pl.BlockSpec((None, D, DH), lambda b, hh: (hh, 0, 0)),
                pl.BlockSpec((None, 1, DH), lambda b, hh: (hh, 0, 0)),
                pl.BlockSpec((None, D, DH), lambda b, hh: (hh, 0, 0)),
                pl.BlockSpec((None, 1, DH), lambda b, hh: (hh, 0, 0)),
                pl.BlockSpec((None, DH, D), lambda b, hh: (hh, 0, 0)),
                pl.BlockSpec((1, D), lambda b, hh: (0, 0)),
            ],
            out_specs=pl.BlockSpec((None, S, D), lambda b, hh: (b, 0, 0)),
            out_shape=jax.ShapeDtypeStruct((B, S, D), f32),
        )(h4, ln1_g[l].reshape(1, D), ln1_b[l].reshape(1, D),
          wqh, bqh, wkh, bkh, wvh, bvh, woh, bo[l].reshape(1, D)).reshape(N, D)

        wg_pad = jnp.pad(gr_w[l], ((0, 0), (0, 128 - G)))
        bg_pad = jnp.pad(gr_b[l].reshape(1, G), ((0, 0), (0, 128 - G)))
        we_pad = jnp.pad(er_w[l], ((0, 0), (0, 128 - E)))
        be_pad = jnp.pad(er_b[l].reshape(1, E), ((0, 0), (0, 128 - E)))
        hn2, meta = pl.pallas_call(
            _router_body,
            out_shape=(jax.ShapeDtypeStruct((N, D), f32),
                       jax.ShapeDtypeStruct((N, 128), f32)),
        )(h, ln2_g[l].reshape(1, D), ln2_b[l].reshape(1, D),
          wg_pad, bg_pad, we_pad, be_pad)

        e1 = meta[:, 0].astype(jnp.int32)
        e2 = meta[:, 1].astype(jnp.int32)
        g1 = meta[:, 2]
        g2 = meta[:, 3]
        flat_e = jnp.stack([e1, e2], 1).reshape(-1)
        order = jnp.argsort(flat_e, stable=True).astype(jnp.int32)
        sorted_e = flat_e[order]
        counts = jnp.sum(flat_e[:, None] == jnp.arange(E)[None, :], 0,
                         dtype=jnp.int32)
        pc = ((counts + TILE - 1) // TILE) * TILE
        zero1 = jnp.zeros((1,), jnp.int32)
        pstart = jnp.concatenate([zero1, jnp.cumsum(pc)[:-1]])
        start = jnp.concatenate([zero1, jnp.cumsum(counts)[:-1]])
        ranks = jnp.arange(N * TOPK, dtype=jnp.int32) - start[sorted_e]
        dest_sorted = pstart[sorted_e] + ranks
        tok_sorted = order // TOPK
        token_src = jnp.zeros((P,), jnp.int32).at[dest_sorted].set(tok_sorted)
        xg = hn2[token_src]
        t_start = jnp.arange(NT, dtype=jnp.int32) * TILE
        te = jnp.clip(jnp.sum(t_start[:, None] >= pstart[None, :], 1) - 1,
                      0, E - 1).astype(jnp.int32)

        y = pl.pallas_call(
            _ffn_body,
            grid_spec=pltpu.PrefetchScalarGridSpec(
                num_scalar_prefetch=1,
                grid=(NT,),
                in_specs=[
                    pl.BlockSpec((TILE, D), lambda t, te_r: (t, 0)),
                    pl.BlockSpec((None, D, DFF), lambda t, te_r: (te_r[t], 0, 0)),
                    pl.BlockSpec((None, 1, DFF), lambda t, te_r: (te_r[t], 0, 0)),
                    pl.BlockSpec((None, DFF, D), lambda t, te_r: (te_r[t], 0, 0)),
                    pl.BlockSpec((None, 1, D), lambda t, te_r: (te_r[t], 0, 0)),
                ],
                out_specs=pl.BlockSpec((TILE, D), lambda t, te_r: (t, 0)),
            ),
            out_shape=jax.ShapeDtypeStruct((P, D), f32),
        )(te, xg, w1[l], b1[l].reshape(E, 1, DFF), w2[l], b2[l].reshape(E, 1, D))

        dest = jnp.zeros((N * TOPK,), jnp.int32).at[order].set(dest_sorted)
        d2 = dest.reshape(N, TOPK)
        h = h + y[d2[:, 0]] * g1[:, None] + y[d2[:, 1]] * g2[:, None]

    hw_pad = jnp.pad(head_w, ((0, 0), (0, 128 - NCLS)))
    bh_pad = jnp.pad(head_b.reshape(1, NCLS), ((0, 0), (0, 128 - NCLS)))
    logits = pl.pallas_call(
        _head_body,
        out_shape=jax.ShapeDtypeStruct((8, 128), f32),
    )(h, lnf_g.reshape(1, D), lnf_b.reshape(1, D), hw_pad, bh_pad)
    return logits[:B, :NCLS]


# speedup vs baseline: 2.4857x; 1.2061x over previous
import jax
import jax.numpy as jnp
from jax import lax
from jax.experimental import pallas as pl
from jax.experimental.pallas import tpu as pltpu

B = 4; IN_CH = 80; T = 512; D = 256; DFF = 1024; G = 4; EPG = 4; E = 16
H = 4; DH = D // H; L = 2; S = 512; TOPK = 2; NCLS = 35
N = B * S
TILE = 128
P = N * TOPK + E * TILE
NT = P // TILE
KIN = 3 * IN_CH


def _gelu(v):
    return 0.5 * v * (1.0 + lax.erf(v * 0.7071067811865476))


def _layernorm(x, g, b):
    m = jnp.mean(x, axis=-1, keepdims=True)
    v = jnp.mean((x - m) ** 2, axis=-1, keepdims=True)
    return (x - m) * lax.rsqrt(v + 1e-5) * g + b


def _stem_body(x_ref, w_ref, b_ref, pos_ref, o_ref):
    y = jnp.dot(x_ref[...], w_ref[...], preferred_element_type=jnp.float32)
    o_ref[...] = _gelu(y + b_ref[...]) + pos_ref[...]


def _attn_body(h_ref, g_ref, bb_ref, wq_ref, bq_ref, wk_ref, bk_ref,
               wv_ref, bv_ref, wo_ref, bo_ref, o_ref):
    hh = h_ref[...]
    hn = _layernorm(hh, g_ref[...], bb_ref[...])
    q = jnp.dot(hn, wq_ref[...], preferred_element_type=jnp.float32) + bq_ref[...]
    k = jnp.dot(hn, wk_ref[...], preferred_element_type=jnp.float32) + bk_ref[...]
    v = jnp.dot(hn, wv_ref[...], preferred_element_type=jnp.float32) + bv_ref[...]
    s = lax.dot_general(q, k, (((1,), (1,)), ((), ())),
                        preferred_element_type=jnp.float32) * (DH ** -0.5)
    s = s - jnp.max(s, axis=-1, keepdims=True)
    e = jnp.exp(s)
    att = e / jnp.sum(e, axis=-1, keepdims=True)
    oh = jnp.dot(att, v, preferred_element_type=jnp.float32)
    part = jnp.dot(oh, wo_ref[...], preferred_element_type=jnp.float32)

    @pl.when(pl.program_id(1) == 0)
    def _init():
        o_ref[...] = hh + part + bo_ref[...]

    @pl.when(pl.program_id(1) != 0)
    def _acc():
        o_ref[...] += part


def _router_body(h_ref, g_ref, b_ref, wg_ref, bg_ref, we_ref, be_ref,
                 hn_ref, meta_ref):
    hn = _layernorm(h_ref[...], g_ref[...], b_ref[...])
    hn_ref[...] = hn
    gl = jnp.dot(hn, wg_ref[...], preferred_element_type=jnp.float32) + bg_ref[...]
    el = jnp.dot(hn, we_ref[...], preferred_element_type=jnp.float32) + be_ref[...]
    lane = lax.broadcasted_iota(jnp.int32, gl.shape, 1)
    neg = jnp.float32(-1e30)
    glm = jnp.where(lane < G, gl, neg)
    ge = jnp.where(lane < G, jnp.exp(glm - jnp.max(glm, -1, keepdims=True)), 0.0)
    gp = ge / jnp.sum(ge, -1, keepdims=True)
    elm = jnp.where(lane < E, el, neg)
    ee = jnp.where(lane < E, jnp.exp(elm - jnp.max(elm, -1, keepdims=True)), 0.0)
    r = lax.broadcasted_iota(jnp.int32, (128, 128), 0)
    c = lax.broadcasted_iota(jnp.int32, (128, 128), 1)
    grpsum = jnp.where((r < E) & (c < E) & ((r // EPG) == (c // EPG)), 1.0, 0.0)
    es = jnp.dot(ee, grpsum, preferred_element_type=jnp.float32,
                 precision=lax.Precision.HIGHEST)
    ep = ee / jnp.where(lane < E, es, 1.0)
    bcast = jnp.where((r < G) & (c < E) & ((c // EPG) == r), 1.0, 0.0)
    gpb = jnp.dot(gp, bcast, preferred_element_type=jnp.float32,
                  precision=lax.Precision.HIGHEST)
    comb = jnp.where(lane < E, gpb * ep, neg)
    big = jnp.int32(1 << 30)
    v1 = jnp.max(comb, -1, keepdims=True)
    a1 = jnp.min(jnp.where(comb == v1, lane, big), -1, keepdims=True)
    comb2 = jnp.where(lane == a1, neg, comb)
    v2 = jnp.max(comb2, -1, keepdims=True)
    a2 = jnp.min(jnp.where(comb2 == v2, lane, big), -1, keepdims=True)
    ssum = v1 + v2 + 1e-9
    meta = jnp.where(lane == 0, a1.astype(jnp.float32), 0.0)
    meta = meta + jnp.where(lane == 1, a2.astype(jnp.float32), 0.0)
    meta = meta + jnp.where(lane == 2, v1 / ssum, 0.0)
    meta = meta + jnp.where(lane == 3, v2 / ssum, 0.0)
    meta_ref[...] = meta


def _ffn_body(te_ref, x_ref, w1_ref, b1_ref, w2_ref, b2_ref, o_ref):
    h1 = _gelu(jnp.dot(x_ref[...], w1_ref[...],
                       preferred_element_type=jnp.float32) + b1_ref[...])
    o_ref[...] = jnp.dot(h1, w2_ref[...],
                         preferred_element_type=jnp.float32) + b2_ref[...]


def _head_body(h_ref, g_ref, b_ref, w_ref, bh_ref, o_ref):
    hh = h_ref[...]
    r = lax.broadcasted_iota(jnp.int32, (8, N), 0)
    c = lax.broadcasted_iota(jnp.int32, (8, N), 1)
    pool = jnp.where((c // S) == r, 1.0 / S, 0.0)
    p = jnp.dot(pool, hh, preferred_element_type=jnp.float32,
                precision=lax.Precision.HIGHEST)
    pn = _layernorm(p, g_ref[...], b_ref[...])
    o_ref[...] = jnp.dot(pn, w_ref[...],
                         preferred_element_type=jnp.float32) + bh_ref[...]


def kernel(x, conv_w, conv_b, pos_emb, ln1_g, ln1_b, wq, bq, wk, bk, wv, bv,
           wo, bo, ln2_g, ln2_b, gr_w, gr_b, er_w, er_b, w1, b1, w2, b2,
           lnf_g, lnf_b, head_w, head_b):
    f32 = jnp.float32

    xt = jnp.transpose(x, (0, 2, 1))
    xp = jnp.pad(xt, ((0, 0), (1, 1), (0, 0)))
    win = jnp.concatenate([xp[:, 0:T], xp[:, 1:T + 1], xp[:, 2:T + 2]], -1)
    xi = jnp.pad(win.reshape(N, KIN), ((0, 0), (0, 256 - KIN)))
    wmat = jnp.pad(conv_w.transpose(2, 1, 0).reshape(KIN, D),
                   ((0, 256 - KIN), (0, 0)))
    pos = jnp.tile(pos_emb[:S], (B, 1))
    h = pl.pallas_call(
        _stem_body,
        out_shape=jax.ShapeDtypeStruct((N, D), f32),
    )(xi, wmat, conv_b.reshape(1, D), pos)

    for l in range(L):
        h4 = h.reshape(B, S, D)
        wqh = wq[l].reshape(D, H, DH).transpose(1, 0, 2)
        wkh = wk[l].reshape(D, H, DH).transpose(1, 0, 2)
        wvh = wv[l].reshape(D, H, DH).transpose(1, 0, 2)
        woh = wo[l].reshape(H, DH, D)
        bqh = bq[l].reshape(H, 1, DH)
        bkh = bk[l].reshape(H, 1, DH)
        bvh = bv[l].reshape(H, 1, DH)
        h = pl.pallas_call(
            _attn_body,
            grid=(B, H),
            in_specs=[
                pl.BlockSpec((None, S, D), lambda b, hh: (b, 0, 0)),
                pl.BlockSpec((1, D), lambda b, hh: (0, 0)),
                pl.BlockSpec((1, D), lambda b, hh: (0, 0)),
                pl.BlockSpec((None, D, DH), lambda b, hh: (hh, 0, 0)),
                pl.BlockSpec((None, 1, DH), lambda b, hh: (hh, 0, 0)),
                pl.BlockSpec((None, D, DH), lambda b, hh: (hh, 0, 0)),
                pl.BlockSpec((None, 1, DH), lambda b, hh: (hh, 0, 0)),
                pl.BlockSpec((None, D, DH), lambda b, hh: (hh, 0, 0)),
                pl.BlockSpec((None, 1, DH), lambda b, hh: (hh, 0, 0)),
                pl.BlockSpec((None, DH, D), lambda b, hh: (hh, 0, 0)),
                pl.BlockSpec((1, D), lambda b, hh: (0, 0)),
            ],
            out_specs=pl.BlockSpec((None, S, D), lambda b, hh: (b, 0, 0)),
            out_shape=jax.ShapeDtypeStruct((B, S, D), f32),
        )(h4, ln1_g[l].reshape(1, D), ln1_b[l].reshape(1, D),
          wqh, bqh, wkh, bkh, wvh, bvh, woh, bo[l].reshape(1, D)).reshape(N, D)

        wg_pad = jnp.pad(gr_w[l], ((0, 0), (0, 128 - G)))
        bg_pad = jnp.pad(gr_b[l].reshape(1, G), ((0, 0), (0, 128 - G)))
        we_pad = jnp.pad(er_w[l], ((0, 0), (0, 128 - E)))
        be_pad = jnp.pad(er_b[l].reshape(1, E), ((0, 0), (0, 128 - E)))
        hn2, meta = pl.pallas_call(
            _router_body,
            out_shape=(jax.ShapeDtypeStruct((N, D), f32),
                       jax.ShapeDtypeStruct((N, 128), f32)),
        )(h, ln2_g[l].reshape(1, D), ln2_b[l].reshape(1, D),
          wg_pad, bg_pad, we_pad, be_pad)

        e1 = meta[:, 0].astype(jnp.int32)
        e2 = meta[:, 1].astype(jnp.int32)
        g1 = meta[:, 2]
        g2 = meta[:, 3]
        flat_e = jnp.stack([e1, e2], 1).reshape(-1)
        oh = (flat_e[:, None] == jnp.arange(E)[None, :]).astype(jnp.int32)
        csum = jnp.cumsum(oh, 0)
        counts = csum[-1]
        rank = jnp.sum(csum * oh, 1) - 1
        pc = ((counts + TILE - 1) // TILE) * TILE
        zero1 = jnp.zeros((1,), jnp.int32)
        pstart = jnp.concatenate([zero1, jnp.cumsum(pc)[:-1]])
        dest = jnp.sum(oh * pstart[None, :], 1) + rank
        token_src = jnp.zeros((P,), jnp.int32).at[dest].set(
            jnp.arange(N * TOPK, dtype=jnp.int32) // TOPK)
        xg = hn2[token_src]
        t_start = jnp.arange(NT, dtype=jnp.int32) * TILE
        te = jnp.clip(jnp.sum(t_start[:, None] >= pstart[None, :], 1) - 1,
                      0, E - 1).astype(jnp.int32)

        y = pl.pallas_call(
            _ffn_body,
            grid_spec=pltpu.PrefetchScalarGridSpec(
                num_scalar_prefetch=1,
                grid=(NT,),
                in_specs=[
                    pl.BlockSpec((TILE, D), lambda t, te_r: (t, 0)),
                    pl.BlockSpec((None, D, DFF), lambda t, te_r: (te_r[t], 0, 0)),
                    pl.BlockSpec((None, 1, DFF), lambda t, te_r: (te_r[t], 0, 0)),
                    pl.BlockSpec((None, DFF, D), lambda t, te_r: (te_r[t], 0, 0)),
                    pl.BlockSpec((None, 1, D), lambda t, te_r: (te_r[t], 0, 0)),
                ],
                out_specs=pl.BlockSpec((TILE, D), lambda t, te_r: (t, 0)),
            ),
            out_shape=jax.ShapeDtypeStruct((P, D), f32),
        )(te, xg, w1[l], b1[l].reshape(E, 1, DFF), w2[l], b2[l].reshape(E, 1, D))

        d2 = dest.reshape(N, TOPK)
        h = h + y[d2[:, 0]] * g1[:, None] + y[d2[:, 1]] * g2[:, None]

    hw_pad = jnp.pad(head_w, ((0, 0), (0, 128 - NCLS)))
    bh_pad = jnp.pad(head_b.reshape(1, NCLS), ((0, 0), (0, 128 - NCLS)))
    logits = pl.pallas_call(
        _head_body,
        out_shape=jax.ShapeDtypeStruct((8, 128), f32),
    )(h, lnf_g.reshape(1, D), lnf_b.reshape(1, D), hw_pad, bh_pad)
    return logits[:B, :NCLS]


# speedup vs baseline: 2.6281x; 1.0573x over previous
import jax
import jax.numpy as jnp
from jax import lax
from jax.experimental import pallas as pl
from jax.experimental.pallas import tpu as pltpu

B = 4; IN_CH = 80; T = 512; D = 256; DFF = 1024; G = 4; EPG = 4; E = 16
H = 4; DH = D // H; L = 2; S = 512; TOPK = 2; NCLS = 35
N = B * S
TILE = 128
P = N * TOPK + E * TILE
NT = P // TILE
KIN = 3 * IN_CH


def _gelu(v):
    return 0.5 * v * (1.0 + lax.erf(v * 0.7071067811865476))


def _layernorm(x, g, b):
    m = jnp.mean(x, axis=-1, keepdims=True)
    v = jnp.mean((x - m) ** 2, axis=-1, keepdims=True)
    return (x - m) * lax.rsqrt(v + 1e-5) * g + b


def _stem_body(x_ref, w_ref, b_ref, pos_ref, o_ref):
    y = jnp.dot(x_ref[...], w_ref[...], preferred_element_type=jnp.float32)
    o_ref[...] = _gelu(y + b_ref[...]) + pos_ref[...]


def _attn_body(h_ref, g_ref, bb_ref, wq_ref, bq_ref, wk_ref, bk_ref,
               wv_ref, bv_ref, wo_ref, bo_ref, o_ref):
    hh = h_ref[...]
    hn = _layernorm(hh, g_ref[...], bb_ref[...])
    q = jnp.dot(hn, wq_ref[...], preferred_element_type=jnp.float32) + bq_ref[...]
    k = jnp.dot(hn, wk_ref[...], preferred_element_type=jnp.float32) + bk_ref[...]
    v = jnp.dot(hn, wv_ref[...], preferred_element_type=jnp.float32) + bv_ref[...]
    s = lax.dot_general(q, k, (((1,), (1,)), ((), ())),
                        preferred_element_type=jnp.float32) * (DH ** -0.5)
    s = s - jnp.max(s, axis=-1, keepdims=True)
    e = jnp.exp(s)
    att = e / jnp.sum(e, axis=-1, keepdims=True)
    oh = jnp.dot(att, v, preferred_element_type=jnp.float32)
    part = jnp.dot(oh, wo_ref[...], preferred_element_type=jnp.float32)

    @pl.when(pl.program_id(1) == 0)
    def _init():
        o_ref[...] = hh + part + bo_ref[...]

    @pl.when(pl.program_id(1) != 0)
    def _acc():
        o_ref[...] += part


def _router_body(h_ref, g_ref, b_ref, wg_ref, bg_ref, we_ref, be_ref,
                 hn_ref, meta_ref):
    hn = _layernorm(h_ref[...], g_ref[...], b_ref[...])
    hn_ref[...] = hn.astype(jnp.bfloat16)
    gl = jnp.dot(hn, wg_ref[...], preferred_element_type=jnp.float32) + bg_ref[...]
    el = jnp.dot(hn, we_ref[...], preferred_element_type=jnp.float32) + be_ref[...]
    lane = lax.broadcasted_iota(jnp.int32, gl.shape, 1)
    neg = jnp.float32(-1e30)
    glm = jnp.where(lane < G, gl, neg)
    ge = jnp.where(lane < G, jnp.exp(glm - jnp.max(glm, -1, keepdims=True)), 0.0)
    gp = ge / jnp.sum(ge, -1, keepdims=True)
    elm = jnp.where(lane < E, el, neg)
    ee = jnp.where(lane < E, jnp.exp(elm - jnp.max(elm, -1, keepdims=True)), 0.0)
    r = lax.broadcasted_iota(jnp.int32, (128, 128), 0)
    c = lax.broadcasted_iota(jnp.int32, (128, 128), 1)
    grpsum = jnp.where((r < E) & (c < E) & ((r // EPG) == (c // EPG)), 1.0, 0.0)
    es = jnp.dot(ee, grpsum, preferred_element_type=jnp.float32,
                 precision=lax.Precision.HIGHEST)
    ep = ee / jnp.where(lane < E, es, 1.0)
    bcast = jnp.where((r < G) & (c < E) & ((c // EPG) == r), 1.0, 0.0)
    gpb = jnp.dot(gp, bcast, preferred_element_type=jnp.float32,
                  precision=lax.Precision.HIGHEST)
    comb = jnp.where(lane < E, gpb * ep, neg)
    big = jnp.int32(1 << 30)
    v1 = jnp.max(comb, -1, keepdims=True)
    a1 = jnp.min(jnp.where(comb == v1, lane, big), -1, keepdims=True)
    comb2 = jnp.where(lane == a1, neg, comb)
    v2 = jnp.max(comb2, -1, keepdims=True)
    a2 = jnp.min(jnp.where(comb2 == v2, lane, big), -1, keepdims=True)
    ssum = v1 + v2 + 1e-9
    meta = jnp.where(lane == 0, a1.astype(jnp.float32), 0.0)
    meta = meta + jnp.where(lane == 1, a2.astype(jnp.float32), 0.0)
    meta = meta + jnp.where(lane == 2, v1 / ssum, 0.0)
    meta = meta + jnp.where(lane == 3, v2 / ssum, 0.0)
    meta_ref[...] = meta


def _ffn_body(te_ref, x_ref, w1_ref, b1_ref, w2_ref, b2_ref, o_ref):
    h1 = _gelu(jnp.dot(x_ref[...], w1_ref[...],
                       preferred_element_type=jnp.float32) + b1_ref[...])
    o_ref[...] = jnp.dot(h1.astype(jnp.bfloat16), w2_ref[...],
                         preferred_element_type=jnp.float32) + b2_ref[...]


def _head_body(h_ref, g_ref, b_ref, w_ref, bh_ref, o_ref):
    hh = h_ref[...]
    r = lax.broadcasted_iota(jnp.int32, (8, N), 0)
    c = lax.broadcasted_iota(jnp.int32, (8, N), 1)
    pool = jnp.where((c // S) == r, 1.0 / S, 0.0)
    p = jnp.dot(pool, hh, preferred_element_type=jnp.float32,
                precision=lax.Precision.HIGHEST)
    pn = _layernorm(p, g_ref[...], b_ref[...])
    o_ref[...] = jnp.dot(pn, w_ref[...],
                         preferred_element_type=jnp.float32) + bh_ref[...]


def kernel(x, conv_w, conv_b, pos_emb, ln1_g, ln1_b, wq, bq, wk, bk, wv, bv,
           wo, bo, ln2_g, ln2_b, gr_w, gr_b, er_w, er_b, w1, b1, w2, b2,
           lnf_g, lnf_b, head_w, head_b):
    f32 = jnp.float32

    xt = jnp.transpose(x, (0, 2, 1))
    xp = jnp.pad(xt, ((0, 0), (1, 1), (0, 0)))
    win = jnp.concatenate([xp[:, 0:T], xp[:, 1:T + 1], xp[:, 2:T + 2]], -1)
    xi = jnp.pad(win.reshape(N, KIN), ((0, 0), (0, 256 - KIN)))
    wmat = jnp.pad(conv_w.transpose(2, 1, 0).reshape(KIN, D),
                   ((0, 256 - KIN), (0, 0)))
    pos = jnp.tile(pos_emb[:S], (B, 1))
    h = pl.pallas_call(
        _stem_body,
        out_shape=jax.ShapeDtypeStruct((N, D), f32),
    )(xi, wmat, conv_b.reshape(1, D), pos)

    for l in range(L):
        h4 = h.reshape(B, S, D)
        wqh = wq[l].reshape(D, H, DH).transpose(1, 0, 2)
        wkh = wk[l].reshape(D, H, DH).transpose(1, 0, 2)
        wvh = wv[l].reshape(D, H, DH).transpose(1, 0, 2)
        woh = wo[l].reshape(H, DH, D)
        bqh = bq[l].reshape(H, 1, DH)
        bkh = bk[l].reshape(H, 1, DH)
        bvh = bv[l].reshape(H, 1, DH)
        h = pl.pallas_call(
            _attn_body,
            grid=(B, H),
            in_specs=[
                pl.BlockSpec((None, S, D), lambda b, hh: (b, 0, 0)),
                pl.BlockSpec((1, D), lambda b, hh: (0, 0)),
                pl.BlockSpec((1, D), lambda b, hh: (0, 0)),
                pl.BlockSpec((None, D, DH), lambda b, hh: (hh, 0, 0)),
                pl.BlockSpec((None, 1, DH), lambda b, hh: (hh, 0, 0)),
                pl.BlockSpec((None, D, DH), lambda b, hh: (hh, 0, 0)),
                pl.BlockSpec((None, 1, DH), lambda b, hh: (hh, 0, 0)),
                pl.BlockSpec((None, D, DH), lambda b, hh: (hh, 0, 0)),
                pl.BlockSpec((None, 1, DH), lambda b, hh: (hh, 0, 0)),
                pl.BlockSpec((None, DH, D), lambda b, hh: (hh, 0, 0)),
                pl.BlockSpec((1, D), lambda b, hh: (0, 0)),
            ],
            out_specs=pl.BlockSpec((None, S, D), lambda b, hh: (b, 0, 0)),
            out_shape=jax.ShapeDtypeStruct((B, S, D), f32),
        )(h4, ln1_g[l].reshape(1, D), ln1_b[l].reshape(1, D),
          wqh, bqh, wkh, bkh, wvh, bvh, woh, bo[l].reshape(1, D)).reshape(N, D)

        wg_pad = jnp.pad(gr_w[l], ((0, 0), (0, 128 - G)))
        bg_pad = jnp.pad(gr_b[l].reshape(1, G), ((0, 0), (0, 128 - G)))
        we_pad = jnp.pad(er_w[l], ((0, 0), (0, 128 - E)))
        be_pad = jnp.pad(er_b[l].reshape(1, E), ((0, 0), (0, 128 - E)))
        hn2, meta = pl.pallas_call(
            _router_body,
            out_shape=(jax.ShapeDtypeStruct((N, D), jnp.bfloat16),
                       jax.ShapeDtypeStruct((N, 128), f32)),
        )(h, ln2_g[l].reshape(1, D), ln2_b[l].reshape(1, D),
          wg_pad, bg_pad, we_pad, be_pad)

        e1 = meta[:, 0].astype(jnp.int32)
        e2 = meta[:, 1].astype(jnp.int32)
        g1 = meta[:, 2]
        g2 = meta[:, 3]
        flat_e = jnp.stack([e1, e2], 1).reshape(-1)
        oh = (flat_e[:, None] == jnp.arange(E)[None, :]).astype(jnp.int32)
        csum = jnp.cumsum(oh, 0)
        counts = csum[-1]
        rank = jnp.sum(csum * oh, 1) - 1
        pc = ((counts + TILE - 1) // TILE) * TILE
        zero1 = jnp.zeros((1,), jnp.int32)
        pstart = jnp.concatenate([zero1, jnp.cumsum(pc)[:-1]])
        dest = jnp.sum(oh * pstart[None, :], 1) + rank
        token_src = jnp.zeros((P,), jnp.int32).at[dest].set(
            jnp.arange(N * TOPK, dtype=jnp.int32) // TOPK)
        xg = hn2[token_src]
        t_start = jnp.arange(NT, dtype=jnp.int32) * TILE
        te = jnp.clip(jnp.sum(t_start[:, None] >= pstart[None, :], 1) - 1,
                      0, E - 1).astype(jnp.int32)

        y = pl.pallas_call(
            _ffn_body,
            grid_spec=pltpu.PrefetchScalarGridSpec(
                num_scalar_prefetch=1,
                grid=(NT,),
                in_specs=[
                    pl.BlockSpec((TILE, D), lambda t, te_r: (t, 0)),
                    pl.BlockSpec((None, D, DFF), lambda t, te_r: (te_r[t], 0, 0)),
                    pl.BlockSpec((None, 1, DFF), lambda t, te_r: (te_r[t], 0, 0)),
                    pl.BlockSpec((None, DFF, D), lambda t, te_r: (te_r[t], 0, 0)),
                    pl.BlockSpec((None, 1, D), lambda t, te_r: (te_r[t], 0, 0)),
                ],
                out_specs=pl.BlockSpec((TILE, D), lambda t, te_r: (t, 0)),
            ),
            out_shape=jax.ShapeDtypeStruct((P, D), f32),
        )(te, xg, w1[l].astype(jnp.bfloat16), b1[l].reshape(E, 1, DFF),
          w2[l].astype(jnp.bfloat16), b2[l].reshape(E, 1, D))

        d2 = dest.reshape(N, TOPK)
        h = h + y[d2[:, 0]] * g1[:, None] + y[d2[:, 1]] * g2[:, None]

    hw_pad = jnp.pad(head_w, ((0, 0), (0, 128 - NCLS)))
    bh_pad = jnp.pad(head_b.reshape(1, NCLS), ((0, 0), (0, 128 - NCLS)))
    logits = pl.pallas_call(
        _head_body,
        out_shape=jax.ShapeDtypeStruct((8, 128), f32),
    )(h, lnf_g.reshape(1, D), lnf_b.reshape(1, D), hw_pad, bh_pad)
    return logits[:B, :NCLS]
